# baseline (speedup 1.0000x reference)
import jax
import jax.numpy as jnp
from jax import lax
from jax.experimental import pallas as pl
from jax.experimental.pallas import tpu as pltpu
from jax.experimental.pallas import tpu_sc as plsc

B, H, S, D = 16, 8, 2048, 128
ROWS = B * H
NTILE = 32
ROWS_PER_TILE = ROWS // NTILE
CH = 512
NCH = S // CH


def _sc_body(k_ref, v_ref, ko_ref, vo_ref, bufA, bufB, sems):
    c = lax.axis_index("c")
    s = lax.axis_index("s")
    tile = c * 16 + s
    base = tile * ROWS_PER_TILE
    bufs = (bufA, bufB)

    for src, dst in ((k_ref, ko_ref), (v_ref, vo_ref)):
        def chunk_slice(i):
            row = base + i // NCH
            off = (i % NCH) * CH
            return (row, pl.ds(off, CH), slice(None))

        def in_copy(i, b):
            return pltpu.make_async_copy(src.at[chunk_slice(i)], bufs[b], sems.at[b])

        def out_copy(i, b):
            return pltpu.make_async_copy(bufs[b], dst.at[chunk_slice(i)], sems.at[2 + b])

        n = ROWS_PER_TILE * NCH
        in_copy(0, 0).start()
        for i in range(n):
            b = i % 2
            nb = 1 - b
            in_copy(i, b).wait()
            if i + 1 < n:
                if i >= 1:
                    out_copy(i - 1, nb).wait()
                in_copy(i + 1, nb).start()
            out_copy(i, b).start()
        out_copy(n - 2, n % 2).wait()
        out_copy(n - 1, (n - 1) % 2).wait()


def kernel(k_val, v_val, k_cache, v_cache):
    k2 = k_val.reshape(ROWS, S, D)
    v2 = v_val.reshape(ROWS, S, D)
    mesh = plsc.VectorSubcoreMesh(core_axis_name="c", subcore_axis_name="s")
    fn = pl.kernel(
        _sc_body,
        out_type=[jax.ShapeDtypeStruct((ROWS, S, D), jnp.float32)] * 2,
        mesh=mesh,
        scratch_types=[
            pltpu.MemorySpace.VMEM((CH, D), jnp.float32),
            pltpu.MemorySpace.VMEM((CH, D), jnp.float32),
            pltpu.SemaphoreType.DMA((4,)),
        ],
    )
    ko, vo = fn(k2, v2)
    return ko.reshape(B, H, S, D), vo.reshape(B, H, S, D)


# speedup vs baseline: 1.0834x; 1.0834x over previous
import jax
import jax.numpy as jnp
from jax import lax
from jax.experimental import pallas as pl
from jax.experimental.pallas import tpu as pltpu
from jax.experimental.pallas import tpu_sc as plsc

B, H, S, D = 16, 8, 2048, 128
ROWS = B * H
NTILE = 32
ROWS_PER_TILE = ROWS // NTILE
CH = 512
NCH = S // CH
BR = 4


def _sc_body(src, dst, bufA, bufB, sems):
    c = lax.axis_index("c")
    s = lax.axis_index("s")
    base = (c * 16 + s) * ROWS_PER_TILE
    bufs = (bufA, bufB)

    def chunk_slice(i):
        row = base + i // NCH
        off = (i % NCH) * CH
        return (row, pl.ds(off, CH), slice(None))

    def in_copy(i, b):
        return pltpu.make_async_copy(src.at[chunk_slice(i)], bufs[b], sems.at[b])

    def out_copy(i, b):
        return pltpu.make_async_copy(bufs[b], dst.at[chunk_slice(i)], sems.at[2 + b])

    n = ROWS_PER_TILE * NCH
    in_copy(0, 0).start()
    for i in range(n):
        b = i % 2
        nb = 1 - b
        in_copy(i, b).wait()
        if i + 1 < n:
            if i >= 1:
                out_copy(i - 1, nb).wait()
            in_copy(i + 1, nb).start()
        out_copy(i, b).start()
    out_copy(n - 2, n % 2).wait()
    out_copy(n - 1, (n - 1) % 2).wait()


def _tc_body(v_ref, vo_ref):
    vo_ref[...] = v_ref[...]


def _sc_copy(x):
    fn = pl.kernel(
        _sc_body,
        out_type=jax.ShapeDtypeStruct((ROWS, S, D), jnp.float32),
        mesh=plsc.VectorSubcoreMesh(core_axis_name="c", subcore_axis_name="s"),
        scratch_types=[
            pltpu.MemorySpace.VMEM((CH, D), jnp.float32),
            pltpu.MemorySpace.VMEM((CH, D), jnp.float32),
            pltpu.SemaphoreType.DMA((4,)),
        ],
    )
    return fn(x)


def _tc_copy(x):
    spec = pl.BlockSpec((BR, S, D), lambda i: (i, 0, 0))
    return pl.pallas_call(
        _tc_body,
        grid=(ROWS // BR,),
        in_specs=[spec],
        out_specs=spec,
        out_shape=jax.ShapeDtypeStruct((ROWS, S, D), jnp.float32),
    )(x)


def kernel(k_val, v_val, k_cache, v_cache):
    k2 = k_val.reshape(ROWS, S, D)
    v2 = v_val.reshape(ROWS, S, D)
    ko = _sc_copy(k2)
    vo = _tc_copy(v2)
    return ko.reshape(B, H, S, D), vo.reshape(B, H, S, D)
